# baseline (speedup 1.0000x reference)
import functools

import jax
import jax.numpy as jnp
from jax import lax
from jax.experimental import pallas as pl
from jax.experimental.pallas import tpu as pltpu
from jax.experimental.pallas import tpu_sc as plsc

_NC = 2
_NS = 16
_NW = _NC * _NS


def _make_segsum(N, D, E):
    e_per_w = E // _NW
    C = 80
    n_chunks = e_per_w // C
    assert n_chunks * C == e_per_w and C % 8 == 0
    rows_per_tile = (N // _NS) // 8 * 8
    tail_r0 = rows_per_tile * _NS
    tail_rows = N - tail_r0
    assert tail_rows % 8 == 0

    K = 25
    n_super = n_chunks // K
    assert n_super * K == n_chunks and K % 2 == 1 and (K * C) % 8 == 0

    mesh = plsc.VectorSubcoreMesh(
        core_axis_name="c", subcore_axis_name="s",
        num_cores=_NC, num_subcores=_NS)

    @functools.partial(
        pl.kernel,
        out_type=jax.ShapeDtypeStruct((_NC, N, D), jnp.float32),
        mesh=mesh,
        scratch_types=[
            pltpu.VMEM((2, K, C), jnp.int32),
            pltpu.VMEM((2, K, C), jnp.int32),
            pltpu.VMEM((2, C, D), jnp.float32),
            pltpu.VMEM_SHARED((N, D), jnp.float32),
            pltpu.SemaphoreType.DMA,
            pltpu.SemaphoreType.DMA,
            pltpu.SemaphoreType.DMA,
            pltpu.SemaphoreType.DMA,
            pltpu.SemaphoreType.DMA,
        ],
    )
    def seg(h_hbm, src_hbm, dst_hbm, zeros_hbm, out_hbm,
            src_v, dst_v, rows_v, acc_sh, sem_a, sem_b, sem_sa, sem_sb,
            sem_i):
        c = lax.axis_index("c")
        s = lax.axis_index("s")
        w = s * _NC + c
        r0 = s * rows_per_tile
        pltpu.sync_copy(src_hbm.at[w, 0], src_v.at[0])
        pltpu.sync_copy(dst_hbm.at[w, 0], dst_v.at[0])
        pltpu.sync_copy(zeros_hbm.at[pl.ds(r0, rows_per_tile)],
                        acc_sh.at[pl.ds(r0, rows_per_tile)])
        if tail_rows:
            @pl.when(s == _NS - 1)
            def _():
                pltpu.sync_copy(zeros_hbm.at[pl.ds(tail_r0, tail_rows)],
                                acc_sh.at[pl.ds(tail_r0, tail_rows)])
        plsc.subcore_barrier()

        for sup in range(n_super):
            sl = sup % 2
            if sup + 1 < n_super:
                pltpu.async_copy(src_hbm.at[w, sup + 1], src_v.at[1 - sl],
                                 sem_i)
                pltpu.async_copy(dst_hbm.at[w, sup + 1], dst_v.at[1 - sl],
                                 sem_i)

            def start_g(i, b, sem):
                pltpu.async_copy(h_hbm.at[src_v.at[sl, i]], rows_v.at[b], sem)

            def wait_g(b, sem):
                pltpu.make_async_copy(h_hbm.at[src_v.at[0, 0]], rows_v.at[b],
                                      sem).wait()

            def start_s(i, b, sem):
                pltpu.async_copy(rows_v.at[b], acc_sh.at[dst_v.at[sl, i]],
                                 sem, add=True)

            def wait_s(b, sem):
                pltpu.make_async_copy(rows_v.at[b],
                                      acc_sh.at[dst_v.at[0, 0]], sem).wait()

            start_g(0, 0, sem_a)
            wait_g(0, sem_a)
            start_s(0, 0, sem_sa)
            start_g(1, 1, sem_b)

            def body(j, carry, sl=sl):
                i = 2 * j
                wait_g(1, sem_b)
                wait_s(0, sem_sa)
                start_s(i + 1, 1, sem_sb)
                start_g(i + 2, 0, sem_a)
                wait_g(0, sem_a)
                wait_s(1, sem_sb)
                start_s(i + 2, 0, sem_sa)

                @pl.when(i + 3 < K)
                def _():
                    start_g(i + 3, 1, sem_b)
                return carry

            lax.fori_loop(0, K // 2, body, 0)
            wait_s(0, sem_sa)
            if sup + 1 < n_super:
                pltpu.make_async_copy(src_hbm.at[w, 0], src_v.at[1 - sl],
                                      sem_i).wait()
                pltpu.make_async_copy(dst_hbm.at[w, 0], dst_v.at[1 - sl],
                                      sem_i).wait()
        plsc.subcore_barrier()
        pltpu.sync_copy(acc_sh.at[pl.ds(r0, rows_per_tile)],
                        out_hbm.at[c, pl.ds(r0, rows_per_tile)])
        if tail_rows:
            @pl.when(s == _NS - 1)
            def _():
                pltpu.sync_copy(acc_sh.at[pl.ds(tail_r0, tail_rows)],
                                out_hbm.at[c, pl.ds(tail_r0, tail_rows)])

    return seg


def _gin_layer(x, agg, scale, W0, b0, g0, bb0, W1, b1, g1, bb1):
    N, Din = x.shape
    Dh = W0.shape[1]

    def body(x_ref, agg_ref, sc_ref, W0_ref, b0_ref, g0_ref, bb0_ref,
             W1_ref, b1_ref, g1_ref, bb1_ref, h_ref, mx_ref):
        xv = x_ref[...]
        z = sc_ref[0, 0] * xv + agg_ref[0] + agg_ref[1]
        z = jnp.dot(z, W0_ref[...], preferred_element_type=jnp.float32)
        z = z + b0_ref[...]
        m = jnp.mean(z, axis=0, keepdims=True)
        v = jnp.mean(jnp.square(z - m), axis=0, keepdims=True)
        z = g0_ref[...] * (z - m) / jnp.sqrt(v + 1e-5) + bb0_ref[...]
        z = jnp.maximum(z, 0.0)
        z = jnp.dot(z, W1_ref[...], preferred_element_type=jnp.float32)
        z = z + b1_ref[...]
        m2 = jnp.mean(z, axis=0, keepdims=True)
        v2 = jnp.mean(jnp.square(z - m2), axis=0, keepdims=True)
        z = g1_ref[...] * (z - m2) / jnp.sqrt(v2 + 1e-5) + bb1_ref[...]
        z = jnp.where(z >= 0.0, z, 0.01 * z)
        h_ref[...] = jnp.concatenate([z, jnp.zeros_like(z)], axis=1)
        mx_ref[...] = jnp.max(xv, axis=0, keepdims=True)

    return pl.pallas_call(
        body,
        out_shape=(jax.ShapeDtypeStruct((N, 2 * Dh), jnp.float32),
                   jax.ShapeDtypeStruct((1, Din), jnp.float32)),
    )(x, agg, scale, W0, b0, g0, bb0, W1, b1, g1, bb1)


def _gin_layer_final(h1, agg, scale, W0, b0, g0, bb0, W1, b1, g1, bb1,
                     xmax, P0W, P0b, P1W, P1b, P2W, P2b):
    N = h1.shape[0]
    Din = W0.shape[0]
    Dh = W0.shape[1]
    Dout = P0W.shape[1]

    def body(x_ref, agg_ref, sc_ref, W0_ref, b0_ref, g0_ref, bb0_ref,
             W1_ref, b1_ref, g1_ref, bb1_ref, xmax_ref,
             P0W_ref, P0b_ref, P1W_ref, P1b_ref, P2W_ref, P2b_ref,
             h_ref, score_ref):
        xv = x_ref[...][:, :Din]
        z = sc_ref[0, 0] * xv + agg_ref[0][:, :Din] + agg_ref[1][:, :Din]
        z = jnp.dot(z, W0_ref[...], preferred_element_type=jnp.float32)
        z = z + b0_ref[...]
        m = jnp.mean(z, axis=0, keepdims=True)
        v = jnp.mean(jnp.square(z - m), axis=0, keepdims=True)
        z = g0_ref[...] * (z - m) / jnp.sqrt(v + 1e-5) + bb0_ref[...]
        z = jnp.maximum(z, 0.0)
        z = jnp.dot(z, W1_ref[...], preferred_element_type=jnp.float32)
        z = z + b1_ref[...]
        m2 = jnp.mean(z, axis=0, keepdims=True)
        v2 = jnp.mean(jnp.square(z - m2), axis=0, keepdims=True)
        z = g1_ref[...] * (z - m2) / jnp.sqrt(v2 + 1e-5) + bb1_ref[...]
        z = jnp.where(z >= 0.0, z, 0.01 * z)
        h_ref[...] = z
        h1max = jnp.max(xv, axis=0, keepdims=True)
        h2max = jnp.max(z, axis=0, keepdims=True)
        score = jnp.dot(xmax_ref[...], P0W_ref[...],
                        preferred_element_type=jnp.float32) + P0b_ref[...]
        score = score + jnp.dot(h1max, P1W_ref[...],
                                preferred_element_type=jnp.float32) + P1b_ref[...]
        score = score + jnp.dot(h2max, P2W_ref[...],
                                preferred_element_type=jnp.float32) + P2b_ref[...]
        score_ref[...] = score

    return pl.pallas_call(
        body,
        out_shape=(jax.ShapeDtypeStruct((N, Dh), jnp.float32),
                   jax.ShapeDtypeStruct((1, Dout), jnp.float32)),
    )(h1, agg, scale, W0, b0, g0, bb0, W1, b1, g1, bb1,
      xmax, P0W, P0b, P1W, P1b, P2W, P2b)


def kernel(x, edge_index, params):
    N, Din = x.shape
    E = edge_index.shape[1]
    e_per_w = E // _NW
    C, K = 80, 25
    n_super = e_per_w // (K * C)
    src = edge_index[0].reshape(_NW, n_super, K, C)
    dst = edge_index[1].reshape(_NW, n_super, K, C)
    L0, L1 = params["layers"]
    P0, P1, P2 = params["pred"]
    Dh = L0["W0"].shape[1]

    zeros_big = jnp.zeros((N, Din), jnp.float32)

    row2 = lambda a: a.reshape(1, -1)
    sc0 = (1.0 + L0["eps"]).reshape(1, 1)
    sc1 = (1.0 + L1["eps"]).reshape(1, 1)

    agg1 = _make_segsum(N, Din, E)(x, src, dst, zeros_big)
    h1, xmax = _gin_layer(
        x, agg1, sc0,
        L0["W0"], row2(L0["b0"]), row2(L0["bn0_g"]), row2(L0["bn0_b"]),
        L0["W1"], row2(L0["b1"]), row2(L0["bn1_g"]), row2(L0["bn1_b"]))

    agg2 = _make_segsum(N, 2 * Dh, E)(h1, src, dst, zeros_big)
    h2, score = _gin_layer_final(
        h1, agg2, sc1,
        L1["W0"], row2(L1["b0"]), row2(L1["bn0_g"]), row2(L1["bn0_b"]),
        L1["W1"], row2(L1["b1"]), row2(L1["bn1_g"]), row2(L1["bn1_b"]),
        xmax, P0["W"], row2(P0["b"]), P1["W"], row2(P1["b"]),
        P2["W"], row2(P2["b"]))
    return (h2, score)


# speedup vs baseline: 1.2140x; 1.2140x over previous
import functools

import jax
import jax.numpy as jnp
from jax import lax
from jax.experimental import pallas as pl
from jax.experimental.pallas import tpu as pltpu
from jax.experimental.pallas import tpu_sc as plsc

_NC = 2
_NS = 16
_NW = _NC * _NS


def _make_segsum(N, D, E):
    e_per_w = E // _NW
    C = 80
    n_chunks = e_per_w // C
    assert n_chunks * C == e_per_w and C % 8 == 0
    rows_per_tile = (N // _NS) // 8 * 8
    tail_r0 = rows_per_tile * _NS
    tail_rows = N - tail_r0
    assert tail_rows % 8 == 0

    K = 25
    n_super = n_chunks // K
    assert n_super * K == n_chunks and K % 2 == 1 and (K * C) % 8 == 0

    mesh = plsc.VectorSubcoreMesh(
        core_axis_name="c", subcore_axis_name="s",
        num_cores=_NC, num_subcores=_NS)

    @functools.partial(
        pl.kernel,
        out_type=jax.ShapeDtypeStruct((_NC, N, D), jnp.float32),
        mesh=mesh,
        scratch_types=[
            pltpu.VMEM((2, K, C), jnp.int32),
            pltpu.VMEM((2, K, C), jnp.int32),
            pltpu.VMEM((2, C, D), jnp.float32),
            pltpu.VMEM_SHARED((N, D), jnp.float32),
            pltpu.SemaphoreType.DMA,
            pltpu.SemaphoreType.DMA,
            pltpu.SemaphoreType.DMA,
        ],
    )
    def seg(h_hbm, src_hbm, dst_hbm, zeros_hbm, out_hbm,
            src_v, dst_v, rows_v, acc_sh, sem_a, sem_b, sem_i):
        c = lax.axis_index("c")
        s = lax.axis_index("s")
        w = s * _NC + c
        r0 = s * rows_per_tile
        pltpu.sync_copy(src_hbm.at[w, 0], src_v.at[0])
        pltpu.sync_copy(dst_hbm.at[w, 0], dst_v.at[0])
        pltpu.sync_copy(zeros_hbm.at[pl.ds(r0, rows_per_tile)],
                        acc_sh.at[pl.ds(r0, rows_per_tile)])
        if tail_rows:
            @pl.when(s == _NS - 1)
            def _():
                pltpu.sync_copy(zeros_hbm.at[pl.ds(tail_r0, tail_rows)],
                                acc_sh.at[pl.ds(tail_r0, tail_rows)])
        plsc.subcore_barrier()

        for sup in range(n_super):
            sl = sup % 2
            if sup + 1 < n_super:
                pltpu.async_copy(src_hbm.at[w, sup + 1], src_v.at[1 - sl],
                                 sem_i)
                pltpu.async_copy(dst_hbm.at[w, sup + 1], dst_v.at[1 - sl],
                                 sem_i)

            def start_g(i, b, sem):
                pltpu.async_copy(h_hbm.at[src_v.at[sl, i]], rows_v.at[b], sem)

            def wait_g(b, sem):
                pltpu.make_async_copy(h_hbm.at[src_v.at[0, 0]], rows_v.at[b],
                                      sem).wait()

            def scat(i, b):
                pltpu.sync_copy(rows_v.at[b], acc_sh.at[dst_v.at[sl, i]],
                                add=True)

            start_g(0, 0, sem_a)
            start_g(1, 1, sem_b)

            def body(j, carry, sl=sl):
                i = 2 * j
                wait_g(0, sem_a)
                scat(i, 0)
                start_g(i + 2, 0, sem_a)
                wait_g(1, sem_b)
                scat(i + 1, 1)

                @pl.when(i + 3 < K)
                def _():
                    start_g(i + 3, 1, sem_b)
                return carry

            lax.fori_loop(0, K // 2, body, 0)
            wait_g(0, sem_a)
            scat(K - 1, 0)
            if sup + 1 < n_super:
                pltpu.make_async_copy(src_hbm.at[w, 0], src_v.at[1 - sl],
                                      sem_i).wait()
                pltpu.make_async_copy(dst_hbm.at[w, 0], dst_v.at[1 - sl],
                                      sem_i).wait()
        plsc.subcore_barrier()
        pltpu.sync_copy(acc_sh.at[pl.ds(r0, rows_per_tile)],
                        out_hbm.at[c, pl.ds(r0, rows_per_tile)])
        if tail_rows:
            @pl.when(s == _NS - 1)
            def _():
                pltpu.sync_copy(acc_sh.at[pl.ds(tail_r0, tail_rows)],
                                out_hbm.at[c, pl.ds(tail_r0, tail_rows)])

    return seg


def _gin_layer(x, agg, scale, W0, b0, g0, bb0, W1, b1, g1, bb1):
    N, Din = x.shape
    Dh = W0.shape[1]

    def body(x_ref, agg_ref, sc_ref, W0_ref, b0_ref, g0_ref, bb0_ref,
             W1_ref, b1_ref, g1_ref, bb1_ref, h_ref, mx_ref):
        xv = x_ref[...]
        z = sc_ref[0, 0] * xv + agg_ref[0] + agg_ref[1]
        z = jnp.dot(z, W0_ref[...], preferred_element_type=jnp.float32)
        z = z + b0_ref[...]
        m = jnp.mean(z, axis=0, keepdims=True)
        v = jnp.mean(jnp.square(z - m), axis=0, keepdims=True)
        z = g0_ref[...] * (z - m) / jnp.sqrt(v + 1e-5) + bb0_ref[...]
        z = jnp.maximum(z, 0.0)
        z = jnp.dot(z, W1_ref[...], preferred_element_type=jnp.float32)
        z = z + b1_ref[...]
        m2 = jnp.mean(z, axis=0, keepdims=True)
        v2 = jnp.mean(jnp.square(z - m2), axis=0, keepdims=True)
        z = g1_ref[...] * (z - m2) / jnp.sqrt(v2 + 1e-5) + bb1_ref[...]
        z = jnp.where(z >= 0.0, z, 0.01 * z)
        h_ref[...] = jnp.concatenate([z, jnp.zeros_like(z)], axis=1)
        mx_ref[...] = jnp.max(xv, axis=0, keepdims=True)

    return pl.pallas_call(
        body,
        out_shape=(jax.ShapeDtypeStruct((N, 2 * Dh), jnp.float32),
                   jax.ShapeDtypeStruct((1, Din), jnp.float32)),
    )(x, agg, scale, W0, b0, g0, bb0, W1, b1, g1, bb1)


def _gin_layer_final(h1, agg, scale, W0, b0, g0, bb0, W1, b1, g1, bb1,
                     xmax, P0W, P0b, P1W, P1b, P2W, P2b):
    N = h1.shape[0]
    Din = W0.shape[0]
    Dh = W0.shape[1]
    Dout = P0W.shape[1]

    def body(x_ref, agg_ref, sc_ref, W0_ref, b0_ref, g0_ref, bb0_ref,
             W1_ref, b1_ref, g1_ref, bb1_ref, xmax_ref,
             P0W_ref, P0b_ref, P1W_ref, P1b_ref, P2W_ref, P2b_ref,
             h_ref, score_ref):
        xv = x_ref[...][:, :Din]
        z = sc_ref[0, 0] * xv + agg_ref[0][:, :Din] + agg_ref[1][:, :Din]
        z = jnp.dot(z, W0_ref[...], preferred_element_type=jnp.float32)
        z = z + b0_ref[...]
        m = jnp.mean(z, axis=0, keepdims=True)
        v = jnp.mean(jnp.square(z - m), axis=0, keepdims=True)
        z = g0_ref[...] * (z - m) / jnp.sqrt(v + 1e-5) + bb0_ref[...]
        z = jnp.maximum(z, 0.0)
        z = jnp.dot(z, W1_ref[...], preferred_element_type=jnp.float32)
        z = z + b1_ref[...]
        m2 = jnp.mean(z, axis=0, keepdims=True)
        v2 = jnp.mean(jnp.square(z - m2), axis=0, keepdims=True)
        z = g1_ref[...] * (z - m2) / jnp.sqrt(v2 + 1e-5) + bb1_ref[...]
        z = jnp.where(z >= 0.0, z, 0.01 * z)
        h_ref[...] = z
        h1max = jnp.max(xv, axis=0, keepdims=True)
        h2max = jnp.max(z, axis=0, keepdims=True)
        score = jnp.dot(xmax_ref[...], P0W_ref[...],
                        preferred_element_type=jnp.float32) + P0b_ref[...]
        score = score + jnp.dot(h1max, P1W_ref[...],
                                preferred_element_type=jnp.float32) + P1b_ref[...]
        score = score + jnp.dot(h2max, P2W_ref[...],
                                preferred_element_type=jnp.float32) + P2b_ref[...]
        score_ref[...] = score

    return pl.pallas_call(
        body,
        out_shape=(jax.ShapeDtypeStruct((N, Dh), jnp.float32),
                   jax.ShapeDtypeStruct((1, Dout), jnp.float32)),
    )(h1, agg, scale, W0, b0, g0, bb0, W1, b1, g1, bb1,
      xmax, P0W, P0b, P1W, P1b, P2W, P2b)


def kernel(x, edge_index, params):
    N, Din = x.shape
    E = edge_index.shape[1]
    e_per_w = E // _NW
    C, K = 80, 25
    n_super = e_per_w // (K * C)
    src = edge_index[0].reshape(_NW, n_super, K, C)
    dst = edge_index[1].reshape(_NW, n_super, K, C)
    L0, L1 = params["layers"]
    P0, P1, P2 = params["pred"]
    Dh = L0["W0"].shape[1]

    zeros_big = jnp.zeros((N, Din), jnp.float32)

    row2 = lambda a: a.reshape(1, -1)
    sc0 = (1.0 + L0["eps"]).reshape(1, 1)
    sc1 = (1.0 + L1["eps"]).reshape(1, 1)

    agg1 = _make_segsum(N, Din, E)(x, src, dst, zeros_big)
    h1, xmax = _gin_layer(
        x, agg1, sc0,
        L0["W0"], row2(L0["b0"]), row2(L0["bn0_g"]), row2(L0["bn0_b"]),
        L0["W1"], row2(L0["b1"]), row2(L0["bn1_g"]), row2(L0["bn1_b"]))

    agg2 = _make_segsum(N, 2 * Dh, E)(h1, src, dst, zeros_big)
    h2, score = _gin_layer_final(
        h1, agg2, sc1,
        L1["W0"], row2(L1["b0"]), row2(L1["bn0_g"]), row2(L1["bn0_b"]),
        L1["W1"], row2(L1["b1"]), row2(L1["bn1_g"]), row2(L1["bn1_b"]),
        xmax, P0["W"], row2(P0["b"]), P1["W"], row2(P1["b"]),
        P2["W"], row2(P2["b"]))
    return (h2, score)


# speedup vs baseline: 1.2587x; 1.0368x over previous
import functools

import jax
import jax.numpy as jnp
from jax import lax
from jax.experimental import pallas as pl
from jax.experimental.pallas import tpu as pltpu
from jax.experimental.pallas import tpu_sc as plsc

_NC = 2
_NS = 16
_NW = _NC * _NS


def _make_segsum(N, D, E, D_out=None):
    if D_out is None:
        D_out = D
    e_per_w = E // _NW
    C = 80
    n_chunks = e_per_w // C
    assert n_chunks * C == e_per_w and C % 8 == 0
    rows_per_tile = (N // _NS) // 8 * 8
    tail_r0 = rows_per_tile * _NS
    tail_rows = N - tail_r0
    assert tail_rows % 8 == 0

    K = 25
    n_super = n_chunks // K
    assert n_super * K == n_chunks and K % 2 == 1 and (K * C) % 8 == 0

    mesh = plsc.VectorSubcoreMesh(
        core_axis_name="c", subcore_axis_name="s",
        num_cores=_NC, num_subcores=_NS)

    @functools.partial(
        pl.kernel,
        out_type=jax.ShapeDtypeStruct((_NC * N, D_out), jnp.float32),
        mesh=mesh,
        scratch_types=[
            pltpu.VMEM((2, K, C), jnp.int32),
            pltpu.VMEM((2, K, C), jnp.int32),
            pltpu.VMEM((2, C, D), jnp.float32),
            pltpu.VMEM_SHARED((N, D_out), jnp.float32),
            pltpu.SemaphoreType.DMA,
            pltpu.SemaphoreType.DMA,
            pltpu.SemaphoreType.DMA,
        ],
    )
    def seg(h_hbm, eidx_hbm, zeros_hbm, out_hbm,
            src_v, dst_v, rows_v, acc_sh, sem_a, sem_b, sem_i):
        c = lax.axis_index("c")
        s = lax.axis_index("s")
        w = s * _NC + c
        r0 = s * rows_per_tile
        pltpu.sync_copy(eidx_hbm.at[0, w, 0], src_v.at[0])
        pltpu.sync_copy(eidx_hbm.at[1, w, 0], dst_v.at[0])
        pltpu.sync_copy(zeros_hbm.at[pl.ds(r0, rows_per_tile)],
                        acc_sh.at[pl.ds(r0, rows_per_tile)])
        if tail_rows:
            @pl.when(s == _NS - 1)
            def _():
                pltpu.sync_copy(zeros_hbm.at[pl.ds(tail_r0, tail_rows)],
                                acc_sh.at[pl.ds(tail_r0, tail_rows)])
        plsc.subcore_barrier()

        for sup in range(n_super):
            sl = sup % 2
            if sup + 1 < n_super:
                pltpu.async_copy(eidx_hbm.at[0, w, sup + 1], src_v.at[1 - sl],
                                 sem_i)
                pltpu.async_copy(eidx_hbm.at[1, w, sup + 1], dst_v.at[1 - sl],
                                 sem_i)

            def start_g(i, b, sem):
                pltpu.async_copy(h_hbm.at[src_v.at[sl, i]], rows_v.at[b], sem)

            def wait_g(b, sem):
                pltpu.make_async_copy(h_hbm.at[src_v.at[0, 0]], rows_v.at[b],
                                      sem).wait()

            def scat(i, b):
                if D_out == D:
                    rows_src = rows_v.at[b]
                else:
                    rows_src = rows_v.at[b, :, pl.ds(0, D_out)]
                pltpu.sync_copy(rows_src, acc_sh.at[dst_v.at[sl, i]],
                                add=True)

            start_g(0, 0, sem_a)
            start_g(1, 1, sem_b)

            def body(j, carry, sl=sl):
                i = 2 * j
                wait_g(0, sem_a)
                scat(i, 0)
                start_g(i + 2, 0, sem_a)
                wait_g(1, sem_b)
                scat(i + 1, 1)

                @pl.when(i + 3 < K)
                def _():
                    start_g(i + 3, 1, sem_b)
                return carry

            lax.fori_loop(0, K // 2, body, 0)
            wait_g(0, sem_a)
            scat(K - 1, 0)
            if sup + 1 < n_super:
                pltpu.make_async_copy(eidx_hbm.at[0, w, 0], src_v.at[1 - sl],
                                      sem_i).wait()
                pltpu.make_async_copy(eidx_hbm.at[1, w, 0], dst_v.at[1 - sl],
                                      sem_i).wait()
        plsc.subcore_barrier()
        pltpu.sync_copy(acc_sh.at[pl.ds(r0, rows_per_tile)],
                        out_hbm.at[pl.ds(c * N + r0, rows_per_tile)])
        if tail_rows:
            @pl.when(s == _NS - 1)
            def _():
                pltpu.sync_copy(acc_sh.at[pl.ds(tail_r0, tail_rows)],
                                out_hbm.at[pl.ds(c * N + tail_r0, tail_rows)])

    return seg


def _gin_layer(x, agg, scale, W0, b0, g0, bb0, W1, b1, g1, bb1):
    N, Din = x.shape
    Dh = W0.shape[1]

    def body(x_ref, agg_ref, sc_ref, W0_ref, b0_ref, g0_ref, bb0_ref,
             W1_ref, b1_ref, g1_ref, bb1_ref, h_ref, mx_ref):
        xv = x_ref[...]
        z = sc_ref[0, 0] * xv + agg_ref[pl.ds(0, N)] + agg_ref[pl.ds(N, N)]
        z = jnp.dot(z, W0_ref[...], preferred_element_type=jnp.float32)
        z = z + b0_ref[...]
        m = jnp.mean(z, axis=0, keepdims=True)
        v = jnp.mean(jnp.square(z - m), axis=0, keepdims=True)
        z = g0_ref[...] * (z - m) / jnp.sqrt(v + 1e-5) + bb0_ref[...]
        z = jnp.maximum(z, 0.0)
        z = jnp.dot(z, W1_ref[...], preferred_element_type=jnp.float32)
        z = z + b1_ref[...]
        m2 = jnp.mean(z, axis=0, keepdims=True)
        v2 = jnp.mean(jnp.square(z - m2), axis=0, keepdims=True)
        z = g1_ref[...] * (z - m2) / jnp.sqrt(v2 + 1e-5) + bb1_ref[...]
        z = jnp.where(z >= 0.0, z, 0.01 * z)
        h_ref[...] = jnp.concatenate([z, jnp.zeros_like(z)], axis=1)
        mx_ref[...] = jnp.max(xv, axis=0, keepdims=True)

    return pl.pallas_call(
        body,
        out_shape=(jax.ShapeDtypeStruct((N, 2 * Dh), jnp.float32),
                   jax.ShapeDtypeStruct((1, Din), jnp.float32)),
    )(x, agg, scale, W0, b0, g0, bb0, W1, b1, g1, bb1)


def _gin_layer_final(h1, agg, scale, W0, b0, g0, bb0, W1, b1, g1, bb1,
                     xmax, P0W, P0b, P1W, P1b, P2W, P2b):
    N = h1.shape[0]
    Din = W0.shape[0]
    Dh = W0.shape[1]
    Dout = P0W.shape[1]

    def body(x_ref, agg_ref, sc_ref, W0_ref, b0_ref, g0_ref, bb0_ref,
             W1_ref, b1_ref, g1_ref, bb1_ref, xmax_ref,
             P0W_ref, P0b_ref, P1W_ref, P1b_ref, P2W_ref, P2b_ref,
             h_ref, score_ref):
        xv = x_ref[...][:, :Din]
        z = (sc_ref[0, 0] * xv + agg_ref[pl.ds(0, N), :Din]
             + agg_ref[pl.ds(N, N), :Din])
        z = jnp.dot(z, W0_ref[...], preferred_element_type=jnp.float32)
        z = z + b0_ref[...]
        m = jnp.mean(z, axis=0, keepdims=True)
        v = jnp.mean(jnp.square(z - m), axis=0, keepdims=True)
        z = g0_ref[...] * (z - m) / jnp.sqrt(v + 1e-5) + bb0_ref[...]
        z = jnp.maximum(z, 0.0)
        z = jnp.dot(z, W1_ref[...], preferred_element_type=jnp.float32)
        z = z + b1_ref[...]
        m2 = jnp.mean(z, axis=0, keepdims=True)
        v2 = jnp.mean(jnp.square(z - m2), axis=0, keepdims=True)
        z = g1_ref[...] * (z - m2) / jnp.sqrt(v2 + 1e-5) + bb1_ref[...]
        z = jnp.where(z >= 0.0, z, 0.01 * z)
        h_ref[...] = z
        h1max = jnp.max(xv, axis=0, keepdims=True)
        h2max = jnp.max(z, axis=0, keepdims=True)
        score = jnp.dot(xmax_ref[...], P0W_ref[...],
                        preferred_element_type=jnp.float32) + P0b_ref[...]
        score = score + jnp.dot(h1max, P1W_ref[...],
                                preferred_element_type=jnp.float32) + P1b_ref[...]
        score = score + jnp.dot(h2max, P2W_ref[...],
                                preferred_element_type=jnp.float32) + P2b_ref[...]
        score_ref[...] = score

    return pl.pallas_call(
        body,
        out_shape=(jax.ShapeDtypeStruct((N, Dh), jnp.float32),
                   jax.ShapeDtypeStruct((1, Dout), jnp.float32)),
    )(h1, agg, scale, W0, b0, g0, bb0, W1, b1, g1, bb1,
      xmax, P0W, P0b, P1W, P1b, P2W, P2b)


def kernel(x, edge_index, params):
    N, Din = x.shape
    E = edge_index.shape[1]
    e_per_w = E // _NW
    C, K = 80, 25
    n_super = e_per_w // (K * C)
    eidx = edge_index.reshape(2, _NW, n_super, K, C)
    L0, L1 = params["layers"]
    P0, P1, P2 = params["pred"]
    Dh = L0["W0"].shape[1]

    zeros_big = jnp.zeros((N, Din), jnp.float32)

    row2 = lambda a: a.reshape(1, -1)
    sc0 = (1.0 + L0["eps"]).reshape(1, 1)
    sc1 = (1.0 + L1["eps"]).reshape(1, 1)

    agg1 = _make_segsum(N, Din, E)(x, eidx, zeros_big)
    h1, xmax = _gin_layer(
        x, agg1, sc0,
        L0["W0"], row2(L0["b0"]), row2(L0["bn0_g"]), row2(L0["bn0_b"]),
        L0["W1"], row2(L0["b1"]), row2(L0["bn1_g"]), row2(L0["bn1_b"]))

    agg2 = _make_segsum(N, 2 * Dh, E)(h1, eidx, zeros_big)
    h2, score = _gin_layer_final(
        h1, agg2, sc1,
        L1["W0"], row2(L1["b0"]), row2(L1["bn0_g"]), row2(L1["bn0_b"]),
        L1["W1"], row2(L1["b1"]), row2(L1["bn1_g"]), row2(L1["bn1_b"]),
        xmax, P0["W"], row2(P0["b"]), P1["W"], row2(P1["b"]),
        P2["W"], row2(P2["b"]))
    return (h2, score)


# speedup vs baseline: 1.2941x; 1.0281x over previous
import functools

import jax
import jax.numpy as jnp
from jax import lax
from jax.experimental import pallas as pl
from jax.experimental.pallas import tpu as pltpu
from jax.experimental.pallas import tpu_sc as plsc

_NC = 2
_NS = 16
_NW = _NC * _NS


def _make_segsum(N, D, E, D_out=None):
    if D_out is None:
        D_out = D
    e_per_w = E // _NW
    C = 80
    n_chunks = e_per_w // C
    assert n_chunks * C == e_per_w and C % 8 == 0
    rows_per_tile = (N // _NS) // 8 * 8
    tail_r0 = rows_per_tile * _NS
    tail_rows = N - tail_r0
    assert tail_rows % 8 == 0

    K = 25
    n_super = n_chunks // K
    assert n_super * K == n_chunks and K % 2 == 1 and (K * C) % 8 == 0

    mesh = plsc.VectorSubcoreMesh(
        core_axis_name="c", subcore_axis_name="s",
        num_cores=_NC, num_subcores=_NS)

    @functools.partial(
        pl.kernel,
        out_type=jax.ShapeDtypeStruct((_NC * N, D_out), jnp.float32),
        mesh=mesh,
        scratch_types=[
            pltpu.VMEM((2, K, C), jnp.int32),
            pltpu.VMEM((2, K, C), jnp.int32),
            pltpu.VMEM((2, C, D), jnp.float32),
            pltpu.VMEM_SHARED((N, D_out), jnp.float32),
            pltpu.SemaphoreType.DMA,
            pltpu.SemaphoreType.DMA,
            pltpu.SemaphoreType.DMA,
        ],
    )
    def seg(h_hbm, eidx_hbm, zeros_hbm, out_hbm,
            src_v, dst_v, rows_v, acc_sh, sem_a, sem_b, sem_i):
        c = lax.axis_index("c")
        s = lax.axis_index("s")
        w = s * _NC + c
        r0 = s * rows_per_tile
        pltpu.sync_copy(eidx_hbm.at[0, w, 0], src_v.at[0])
        pltpu.sync_copy(eidx_hbm.at[1, w, 0], dst_v.at[0])
        pltpu.sync_copy(zeros_hbm.at[pl.ds(r0, rows_per_tile)],
                        acc_sh.at[pl.ds(r0, rows_per_tile)])
        if tail_rows:
            @pl.when(s == _NS - 1)
            def _():
                pltpu.sync_copy(zeros_hbm.at[pl.ds(tail_r0, tail_rows)],
                                acc_sh.at[pl.ds(tail_r0, tail_rows)])
        plsc.subcore_barrier()

        def start_g(sl, i, b):
            sem = sem_a if b == 0 else sem_b
            pltpu.async_copy(h_hbm.at[src_v.at[sl, i]], rows_v.at[b], sem)

        def wait_g(b):
            sem = sem_a if b == 0 else sem_b
            pltpu.make_async_copy(h_hbm.at[src_v.at[0, 0]], rows_v.at[b],
                                  sem).wait()

        def scat(sl, i, b):
            if D_out == D:
                rows_src = rows_v.at[b]
            else:
                rows_src = rows_v.at[b, :, pl.ds(0, D_out)]
            pltpu.sync_copy(rows_src, acc_sh.at[dst_v.at[sl, i]], add=True)

        start_g(0, 0, 0)
        start_g(0, 1, 1)
        for sup in range(n_super):
            sl = sup % 2
            bufm = lambda i, sup=sup: (sup * K + i) % 2
            if sup + 1 < n_super:
                pltpu.async_copy(eidx_hbm.at[0, w, sup + 1], src_v.at[1 - sl],
                                 sem_i)
                pltpu.async_copy(eidx_hbm.at[1, w, sup + 1], dst_v.at[1 - sl],
                                 sem_i)

            b0 = bufm(0)

            def body(j, carry, sl=sl, b0=b0):
                i = 2 * j
                wait_g(b0)
                scat(sl, i, b0)
                start_g(sl, i + 2, b0)
                wait_g(1 - b0)
                scat(sl, i + 1, 1 - b0)
                start_g(sl, i + 3, 1 - b0)
                return carry

            lax.fori_loop(0, (K - 3) // 2, body, 0)
            wait_g(bufm(K - 3))
            scat(sl, K - 3, bufm(K - 3))
            start_g(sl, K - 1, bufm(K - 1))
            if sup + 1 < n_super:
                pltpu.make_async_copy(eidx_hbm.at[0, w, 0], src_v.at[1 - sl],
                                      sem_i).wait()
                pltpu.make_async_copy(eidx_hbm.at[1, w, 0], dst_v.at[1 - sl],
                                      sem_i).wait()
                start_g(1 - sl, 0, bufm(K - 3))
                wait_g(bufm(K - 2))
                scat(sl, K - 2, bufm(K - 2))
                start_g(1 - sl, 1, bufm(K - 2))
                wait_g(bufm(K - 1))
                scat(sl, K - 1, bufm(K - 1))
            else:
                wait_g(bufm(K - 2))
                scat(sl, K - 2, bufm(K - 2))
                wait_g(bufm(K - 1))
                scat(sl, K - 1, bufm(K - 1))
        plsc.subcore_barrier()
        pltpu.sync_copy(acc_sh.at[pl.ds(r0, rows_per_tile)],
                        out_hbm.at[pl.ds(c * N + r0, rows_per_tile)])
        if tail_rows:
            @pl.when(s == _NS - 1)
            def _():
                pltpu.sync_copy(acc_sh.at[pl.ds(tail_r0, tail_rows)],
                                out_hbm.at[pl.ds(c * N + tail_r0, tail_rows)])

    return seg


def _gin_layer(x, agg, scale, W0, b0, g0, bb0, W1, b1, g1, bb1):
    N, Din = x.shape
    Dh = W0.shape[1]

    def body(x_ref, agg_ref, sc_ref, W0_ref, b0_ref, g0_ref, bb0_ref,
             W1_ref, b1_ref, g1_ref, bb1_ref, h_ref, mx_ref):
        xv = x_ref[...]
        z = sc_ref[0, 0] * xv + agg_ref[pl.ds(0, N)] + agg_ref[pl.ds(N, N)]
        z = jnp.dot(z, W0_ref[...], preferred_element_type=jnp.float32)
        z = z + b0_ref[...]
        m = jnp.mean(z, axis=0, keepdims=True)
        v = jnp.mean(jnp.square(z - m), axis=0, keepdims=True)
        z = g0_ref[...] * (z - m) / jnp.sqrt(v + 1e-5) + bb0_ref[...]
        z = jnp.maximum(z, 0.0)
        z = jnp.dot(z, W1_ref[...], preferred_element_type=jnp.float32)
        z = z + b1_ref[...]
        m2 = jnp.mean(z, axis=0, keepdims=True)
        v2 = jnp.mean(jnp.square(z - m2), axis=0, keepdims=True)
        z = g1_ref[...] * (z - m2) / jnp.sqrt(v2 + 1e-5) + bb1_ref[...]
        z = jnp.where(z >= 0.0, z, 0.01 * z)
        h_ref[...] = jnp.concatenate([z, jnp.zeros_like(z)], axis=1)
        mx_ref[...] = jnp.max(xv, axis=0, keepdims=True)

    return pl.pallas_call(
        body,
        out_shape=(jax.ShapeDtypeStruct((N, 2 * Dh), jnp.float32),
                   jax.ShapeDtypeStruct((1, Din), jnp.float32)),
    )(x, agg, scale, W0, b0, g0, bb0, W1, b1, g1, bb1)


def _gin_layer_final(h1, agg, scale, W0, b0, g0, bb0, W1, b1, g1, bb1,
                     xmax, P0W, P0b, P1W, P1b, P2W, P2b):
    N = h1.shape[0]
    Din = W0.shape[0]
    Dh = W0.shape[1]
    Dout = P0W.shape[1]

    def body(x_ref, agg_ref, sc_ref, W0_ref, b0_ref, g0_ref, bb0_ref,
             W1_ref, b1_ref, g1_ref, bb1_ref, xmax_ref,
             P0W_ref, P0b_ref, P1W_ref, P1b_ref, P2W_ref, P2b_ref,
             h_ref, score_ref):
        xv = x_ref[...][:, :Din]
        z = (sc_ref[0, 0] * xv + agg_ref[pl.ds(0, N), :Din]
             + agg_ref[pl.ds(N, N), :Din])
        z = jnp.dot(z, W0_ref[...], preferred_element_type=jnp.float32)
        z = z + b0_ref[...]
        m = jnp.mean(z, axis=0, keepdims=True)
        v = jnp.mean(jnp.square(z - m), axis=0, keepdims=True)
        z = g0_ref[...] * (z - m) / jnp.sqrt(v + 1e-5) + bb0_ref[...]
        z = jnp.maximum(z, 0.0)
        z = jnp.dot(z, W1_ref[...], preferred_element_type=jnp.float32)
        z = z + b1_ref[...]
        m2 = jnp.mean(z, axis=0, keepdims=True)
        v2 = jnp.mean(jnp.square(z - m2), axis=0, keepdims=True)
        z = g1_ref[...] * (z - m2) / jnp.sqrt(v2 + 1e-5) + bb1_ref[...]
        z = jnp.where(z >= 0.0, z, 0.01 * z)
        h_ref[...] = z
        h1max = jnp.max(xv, axis=0, keepdims=True)
        h2max = jnp.max(z, axis=0, keepdims=True)
        score = jnp.dot(xmax_ref[...], P0W_ref[...],
                        preferred_element_type=jnp.float32) + P0b_ref[...]
        score = score + jnp.dot(h1max, P1W_ref[...],
                                preferred_element_type=jnp.float32) + P1b_ref[...]
        score = score + jnp.dot(h2max, P2W_ref[...],
                                preferred_element_type=jnp.float32) + P2b_ref[...]
        score_ref[...] = score

    return pl.pallas_call(
        body,
        out_shape=(jax.ShapeDtypeStruct((N, Dh), jnp.float32),
                   jax.ShapeDtypeStruct((1, Dout), jnp.float32)),
    )(h1, agg, scale, W0, b0, g0, bb0, W1, b1, g1, bb1,
      xmax, P0W, P0b, P1W, P1b, P2W, P2b)


def kernel(x, edge_index, params):
    N, Din = x.shape
    E = edge_index.shape[1]
    e_per_w = E // _NW
    C, K = 80, 25
    n_super = e_per_w // (K * C)
    eidx = edge_index.reshape(2, _NW, n_super, K, C)
    L0, L1 = params["layers"]
    P0, P1, P2 = params["pred"]
    Dh = L0["W0"].shape[1]

    zeros_big = jnp.zeros((N, Din), jnp.float32)

    row2 = lambda a: a.reshape(1, -1)
    sc0 = (1.0 + L0["eps"]).reshape(1, 1)
    sc1 = (1.0 + L1["eps"]).reshape(1, 1)

    agg1 = _make_segsum(N, Din, E)(x, eidx, zeros_big)
    h1, xmax = _gin_layer(
        x, agg1, sc0,
        L0["W0"], row2(L0["b0"]), row2(L0["bn0_g"]), row2(L0["bn0_b"]),
        L0["W1"], row2(L0["b1"]), row2(L0["bn1_g"]), row2(L0["bn1_b"]))

    agg2 = _make_segsum(N, 2 * Dh, E)(h1, eidx, zeros_big)
    h2, score = _gin_layer_final(
        h1, agg2, sc1,
        L1["W0"], row2(L1["b0"]), row2(L1["bn0_g"]), row2(L1["bn0_b"]),
        L1["W1"], row2(L1["b1"]), row2(L1["bn1_g"]), row2(L1["bn1_b"]),
        xmax, P0["W"], row2(P0["b"]), P1["W"], row2(P1["b"]),
        P2["W"], row2(P2["b"]))
    return (h2, score)
